# baseline (speedup 1.0000x reference)
import jax
import jax.numpy as jnp
from jax.experimental import pallas as pl
from jax.experimental.pallas import tpu as pltpu
from jax.experimental.pallas import tpu_sc as plsc

_INTERPRET = False

B, S, D, H, F, E = 1, 2048, 1024, 16, 4096, 8
HD = D // H
HALF = HD // 2
SBLK = 256
NSB = S // SBLK
QBLK = 512
EPAD = 128
FBLK = F // 2
NEG = -1e30
M = 256
NB = (2 * S) // M + E
RMAX = NB * M
_vector_mesh = None


def _get_mesh():
    global _vector_mesh
    if _vector_mesh is None:
        _vector_mesh = plsc.VectorSubcoreMesh(core_axis_name="c",
                                              subcore_axis_name="s")
    return _vector_mesh


def _ln_f32(xb, g, b):
    m = jnp.mean(xb, axis=-1, keepdims=True)
    v = jnp.mean((xb - m) ** 2, axis=-1, keepdims=True)
    return (xb - m) / jnp.sqrt(v + 1e-5) * g + b


def _qkv_kernel(x_ref, inwT_ref, inb_ref, g_ref, b_ref, inv_ref, pswap_ref,
                q2_ref, k2_ref, v2_ref):
    s = pl.program_id(0)
    xb = x_ref[...]
    h = _ln_f32(xb, g_ref[...], b_ref[...])
    w = inwT_ref[...]
    qkv = jnp.dot(h.astype(jnp.bfloat16), w,
                  preferred_element_type=jnp.float32) + inb_ref[...]
    q, k, v = qkv[:, :D], qkv[:, D:2 * D], qkv[:, 2 * D:]
    pos = (s * SBLK
           + jax.lax.broadcasted_iota(jnp.int32, (SBLK, HALF), 0)).astype(jnp.float32)
    ang = pos * inv_ref[...]
    c32, s32 = jnp.cos(ang), jnp.sin(ang)
    c64 = jnp.concatenate([c32, c32], axis=1)
    s64 = jnp.concatenate([s32, s32], axis=1)
    cf = jnp.concatenate([c64] * H, axis=1)
    sf = jnp.concatenate([s64] * H, axis=1)
    psw = pswap_ref[...]
    qsw = jnp.dot(q.astype(jnp.bfloat16), psw, preferred_element_type=jnp.float32)
    ksw = jnp.dot(k.astype(jnp.bfloat16), psw, preferred_element_type=jnp.float32)
    rq = (q * cf + qsw * sf).astype(jnp.bfloat16)
    rk = (k * cf + ksw * sf).astype(jnp.bfloat16)
    q2_ref[...] = (jnp.dot(rq, w[:, :D], preferred_element_type=jnp.float32)
                   + inb_ref[:, :D]).astype(jnp.bfloat16)
    k2_ref[...] = (jnp.dot(rk, w[:, D:2 * D], preferred_element_type=jnp.float32)
                   + inb_ref[:, D:2 * D]).astype(jnp.bfloat16)
    v2_ref[...] = (jnp.dot(v.astype(jnp.bfloat16), w[:, 2 * D:],
                           preferred_element_type=jnp.float32)
                   + inb_ref[:, 2 * D:]).astype(jnp.bfloat16)


def _attn_kernel(q_ref, k_ref, v_ref, o_ref):
    qb = q_ref[...]
    kb = k_ref[...]
    vb = v_ref[...]
    outs = []
    for hh in range(2):
        q1 = qb[:, hh * HD:(hh + 1) * HD]
        k1 = kb[:, hh * HD:(hh + 1) * HD]
        v1 = vb[:, hh * HD:(hh + 1) * HD]
        sc = jax.lax.dot_general(q1, k1, (((1,), (1,)), ((), ())),
                                 preferred_element_type=jnp.float32) * 0.125
        m = jnp.max(sc, axis=1, keepdims=True)
        p = jnp.exp(sc - m)
        z = jnp.sum(p, axis=1, keepdims=True)
        pb = (p / z).astype(jnp.bfloat16)
        outs.append(jnp.dot(pb, v1, preferred_element_type=jnp.float32))
    o_ref[...] = jnp.concatenate(outs, axis=1).astype(jnp.bfloat16)


def _post_kernel(ctx_ref, x_ref, outwT_ref, outb_ref, g2_ref, b2_ref, rwT_ref,
                 x1_ref, h2_ref, w_ref):
    ctx = ctx_ref[...]
    attn = jnp.dot(ctx, outwT_ref[...],
                   preferred_element_type=jnp.float32) + outb_ref[...]
    x1 = x_ref[...] + attn
    x1_ref[...] = x1
    h2 = _ln_f32(x1, g2_ref[...], b2_ref[...])
    h2_ref[...] = h2.astype(jnp.bfloat16)
    logits = jnp.dot(h2, rwT_ref[...], preferred_element_type=jnp.float32)
    lane = jax.lax.broadcasted_iota(jnp.int32, (SBLK, EPAD), 1)
    l = jnp.where(lane < E, logits, NEG)
    m1 = jnp.max(l, axis=1, keepdims=True)
    i1 = jnp.min(jnp.where(l == m1, lane, EPAD), axis=1, keepdims=True)
    l2 = jnp.where(lane == i1, NEG, l)
    m2 = jnp.max(l2, axis=1, keepdims=True)
    i2 = jnp.min(jnp.where(l2 == m2, lane, EPAD), axis=1, keepdims=True)
    ex = jnp.where(lane < E, jnp.exp(l - m1), 0.0)
    zz = jnp.sum(ex, axis=1, keepdims=True)
    p = ex / zz
    w_ref[...] = jnp.where((lane == i1) | (lane == i2), p, -1.0)


def _route_kernel(w_ref, pk_ref):
    w = w_ref[...]
    lane = jax.lax.broadcasted_iota(jnp.int32, (S, EPAD), 1)
    sub = jax.lax.broadcasted_iota(jnp.int32, (S, EPAD), 0)
    sel = w >= 0.0
    oh = jnp.where(sel, 1.0, 0.0)
    r_i = jax.lax.broadcasted_iota(jnp.int32, (M, M), 0)
    c_i = jax.lax.broadcasted_iota(jnp.int32, (M, M), 1)
    tri = jnp.where(r_i > c_i, 1.0, 0.0).astype(jnp.bfloat16)
    carry = jnp.zeros((1, EPAD), jnp.float32)
    ranks = []
    for blk in range(S // M):
        ob = oh[blk * M:(blk + 1) * M, :]
        rb = jax.lax.dot_general(tri, ob.astype(jnp.bfloat16),
                                 (((1,), (0,)), ((), ())),
                                 preferred_element_type=jnp.float32)
        ranks.append(rb + carry)
        carry = carry + jnp.sum(ob, axis=0, keepdims=True)
    rank = jnp.concatenate(ranks, axis=0)
    counts = carry
    pc = jnp.ceil(counts * (1.0 / M)) * M
    off_sc = []
    run = jnp.float32(0.0)
    for e in range(E):
        off_sc.append(run)
        run = run + pc[0, e]
    nact = run * (1.0 / M)
    off_vec = jnp.zeros((1, EPAD), jnp.float32)
    end_vec = jnp.full((1, EPAD), jnp.float32(NEG * -1.0))
    for e in range(E):
        lm = (lane[:1, :] == e)
        off_vec = jnp.where(lm, off_sc[e], off_vec)
        end_vec = jnp.where(lm, off_sc[e] + pc[0, e], end_vec)
    val = rank + off_vec
    lane_sel = jnp.where(sel, lane, EPAD)
    e_lo = jnp.min(lane_sel, axis=1, keepdims=True)
    lane_sel2 = jnp.where(sel, lane, -1)
    e_hi = jnp.max(lane_sel2, axis=1, keepdims=True)
    d_lo = jnp.sum(jnp.where(lane == e_lo, val, 0.0), axis=1, keepdims=True)
    d_hi = jnp.sum(jnp.where(lane == e_hi, val, 0.0), axis=1, keepdims=True)
    w_lo = jnp.sum(jnp.where(lane == e_lo, jnp.maximum(w, 0.0), 0.0),
                   axis=1, keepdims=True)
    w_hi = jnp.sum(jnp.where(lane == e_hi, jnp.maximum(w, 0.0), 0.0),
                   axis=1, keepdims=True)
    bvals = (sub[:, :1] * M).astype(jnp.float32)
    ge = (bvals >= end_vec) & (lane[:1, :] < E)
    be = jnp.sum(jnp.where(ge, 1.0, 0.0), axis=1, keepdims=True)
    be = jnp.minimum(be, E - 1)
    pk = jnp.where(lane == 0, d_lo, 0.0)
    pk = pk + jnp.where(lane == 1, d_hi, 0.0)
    pk = pk + jnp.where(lane == 2, w_lo, 0.0)
    pk = pk + jnp.where(lane == 3, w_hi, 0.0)
    pk = pk + jnp.where(lane == 4, be, 0.0)
    pk = pk + jnp.where(lane == 5, nact, 0.0)
    pk_ref[...] = pk


def _gmm_kernel(be_ref, na_ref, xd_ref, gw_ref, vw_ref, ow_ref,
                gb_ref, vb_ref, ob_ref, y_ref):
    f = pl.program_id(0)
    b = pl.program_id(1)

    @pl.when(b < na_ref[0])
    def _():
        xb = xd_ref[...]
        g = jax.lax.dot_general(xb, gw_ref[0], (((1,), (1,)), ((), ())),
                                preferred_element_type=jnp.float32) + gb_ref[0, 0]
        v = jax.lax.dot_general(xb, vw_ref[0], (((1,), (1,)), ((), ())),
                                preferred_element_type=jnp.float32) + vb_ref[0, 0]
        hh = (v * (g * jax.nn.sigmoid(g))).astype(jnp.bfloat16)
        o = jax.lax.dot_general(hh, ow_ref[0], (((1,), (1,)), ((), ())),
                                preferred_element_type=jnp.float32)
        ob = ob_ref[0, 0][None, :] * (f == 0).astype(jnp.float32)
        y_ref[0] = (o + ob).astype(jnp.bfloat16)


def _combine_kernel(x1_ref, wlo_ref, whi_ref, ya_ref, yb_ref, yc_ref, yd_ref,
                    out_ref):
    f32 = jnp.float32
    ylo = ya_ref[...].astype(f32) + yc_ref[...].astype(f32)
    yhi = yb_ref[...].astype(f32) + yd_ref[...].astype(f32)
    out_ref[...] = x1_ref[...] + wlo_ref[...] * ylo + whi_ref[...] * yhi


CHUNK = 128
RPT = 4
SCW = 128


def _subrow_idx(rows_idx):
    return (rows_idx[..., None] * RPT
            + jnp.arange(RPT, dtype=jnp.int32)).reshape(1, -1)


def _to_i32_rows(a2d):
    n = a2d.shape[0]
    return jax.lax.bitcast_convert_type(
        a2d.reshape(n, D // 2, 2), jnp.int32).reshape(n * RPT, CHUNK)


def _from_i32_rows(a2d, n):
    return jax.lax.bitcast_convert_type(
        a2d.reshape(n, D // 2), jnp.bfloat16).reshape(n, D)


def _sc_scatter(h2b, dT):
    h2r = _to_i32_rows(h2b)
    idx = _subrow_idx(dT)
    nblk = S * RPT // SCW

    @pl.kernel(out_type=jax.ShapeDtypeStruct((RMAX * RPT, CHUNK), jnp.int32),
               mesh=_get_mesh())
    def k(h2_hbm, i_hbm, o_hbm):
        def body(x_vmem, i_vmem):
            pltpu.sync_copy(x_vmem, o_hbm.at[i_vmem.at[0]])
        pltpu.emit_pipeline(
            body,
            grid=(2 * nblk,),
            in_specs=[pl.BlockSpec((SCW, CHUNK),
                                   lambda i: (jax.lax.rem(i, nblk), 0)),
                      pl.BlockSpec((1, SCW), lambda i: (0, i))],
            out_specs=[],
            core_axis_name=("c", "s"),
            dimension_semantics=(pltpu.PARALLEL,),
        )(h2_hbm, i_hbm)
    return _from_i32_rows(k(h2r, idx), RMAX)


def _sc_gather(yflat, gidx):
    yr = _to_i32_rows(yflat)
    idx = _subrow_idx(gidx)

    @pl.kernel(out_type=jax.ShapeDtypeStruct((4 * S * RPT, CHUNK), jnp.int32),
               mesh=_get_mesh())
    def k(y_hbm, i_hbm, o_hbm):
        def body(i_vmem, o_vmem):
            pltpu.sync_copy(y_hbm.at[i_vmem.at[0]], o_vmem)
        pltpu.emit_pipeline(
            body,
            grid=(4 * S * RPT // SCW,),
            in_specs=[pl.BlockSpec((1, SCW), lambda i: (0, i))],
            out_specs=[pl.BlockSpec((SCW, CHUNK), lambda i: (i, 0))],
            core_axis_name=("c", "s"),
            dimension_semantics=(pltpu.PARALLEL,),
        )(i_hbm, o_hbm)
    return _from_i32_rows(k(yr, idx), 4 * S)


def _cparams(sem):
    return pltpu.CompilerParams(dimension_semantics=sem)


def kernel(x, ln1_g, ln1_b, in_w, in_b, out_w, out_b, ln2_g, ln2_b,
           router_w, gate_w, gate_b, val_w, val_b, wo_w, wo_b):
    f32, bf16 = jnp.float32, jnp.bfloat16
    x2 = x.reshape(S, D)
    inwT = in_w.T.astype(bf16)
    inb = in_b.reshape(1, 3 * D)
    g1 = ln1_g.reshape(1, D)
    b1 = ln1_b.reshape(1, D)
    inv = (1.0 / (10000.0 ** (jnp.arange(HALF, dtype=f32) / HALF))).reshape(1, HALF)
    eye = jnp.eye(HALF, dtype=f32)
    zer = jnp.zeros((HALF, HALF), f32)
    p64 = jnp.concatenate([
        jnp.concatenate([zer, eye], axis=1),
        jnp.concatenate([-eye, zer], axis=1)], axis=0)
    pswap = jnp.kron(jnp.eye(H, dtype=f32), p64).astype(bf16)

    q2, k2, v2 = pl.pallas_call(
        _qkv_kernel,
        grid=(NSB,),
        in_specs=[
            pl.BlockSpec((SBLK, D), lambda s: (s, 0)),
            pl.BlockSpec((D, 3 * D), lambda s: (0, 0)),
            pl.BlockSpec((1, 3 * D), lambda s: (0, 0)),
            pl.BlockSpec((1, D), lambda s: (0, 0)),
            pl.BlockSpec((1, D), lambda s: (0, 0)),
            pl.BlockSpec((1, HALF), lambda s: (0, 0)),
            pl.BlockSpec((D, D), lambda s: (0, 0)),
        ],
        out_specs=[pl.BlockSpec((SBLK, D), lambda s: (s, 0))] * 3,
        out_shape=[jax.ShapeDtypeStruct((S, D), bf16)] * 3,
        compiler_params=_cparams(("arbitrary",)),
        interpret=_INTERPRET,
    )(x2, inwT, inb, g1, b1, inv, pswap)

    ctx = pl.pallas_call(
        _attn_kernel,
        grid=(H // 2, S // QBLK),
        in_specs=[
            pl.BlockSpec((QBLK, 2 * HD), lambda hp, sq: (sq, hp)),
            pl.BlockSpec((S, 2 * HD), lambda hp, sq: (0, hp)),
            pl.BlockSpec((S, 2 * HD), lambda hp, sq: (0, hp)),
        ],
        out_specs=pl.BlockSpec((QBLK, 2 * HD), lambda hp, sq: (sq, hp)),
        out_shape=jax.ShapeDtypeStruct((S, D), bf16),
        compiler_params=_cparams(("arbitrary", "arbitrary")),
        interpret=_INTERPRET,
    )(q2, k2, v2)

    outwT = out_w.T.astype(bf16)
    outb = out_b.reshape(1, D)
    g2 = ln2_g.reshape(1, D)
    b2 = ln2_b.reshape(1, D)
    rwT = jnp.zeros((D, EPAD), f32).at[:, :E].set(router_w.T)

    x1, h2b, w = pl.pallas_call(
        _post_kernel,
        grid=(NSB,),
        in_specs=[
            pl.BlockSpec((SBLK, D), lambda s: (s, 0)),
            pl.BlockSpec((SBLK, D), lambda s: (s, 0)),
            pl.BlockSpec((D, D), lambda s: (0, 0)),
            pl.BlockSpec((1, D), lambda s: (0, 0)),
            pl.BlockSpec((1, D), lambda s: (0, 0)),
            pl.BlockSpec((1, D), lambda s: (0, 0)),
            pl.BlockSpec((D, EPAD), lambda s: (0, 0)),
        ],
        out_specs=[
            pl.BlockSpec((SBLK, D), lambda s: (s, 0)),
            pl.BlockSpec((SBLK, D), lambda s: (s, 0)),
            pl.BlockSpec((SBLK, EPAD), lambda s: (s, 0)),
        ],
        out_shape=[
            jax.ShapeDtypeStruct((S, D), f32),
            jax.ShapeDtypeStruct((S, D), bf16),
            jax.ShapeDtypeStruct((S, EPAD), f32),
        ],
        compiler_params=_cparams(("arbitrary",)),
        interpret=_INTERPRET,
    )(ctx, x2, outwT, outb, g2, b2, rwT)

    pk = pl.pallas_call(
        _route_kernel,
        in_specs=[pl.BlockSpec((S, EPAD), lambda: (0, 0))],
        out_specs=pl.BlockSpec((S, EPAD), lambda: (0, 0)),
        out_shape=jax.ShapeDtypeStruct((S, EPAD), f32),
        interpret=_INTERPRET,
    )(w)

    dT = pk[:, 0:2].T.astype(jnp.int32)
    gidx = jnp.concatenate([dT, dT + RMAX], axis=0)
    wlo = pk[:, 2:3]
    whi = pk[:, 3:4]
    be = pk[:NB, 4].astype(jnp.int32)
    nact = pk[0:1, 5].astype(jnp.int32)

    xdisp = _sc_scatter(h2b, dT)

    gwb = gate_w.astype(bf16)
    vwb = val_w.astype(bf16)
    owb = wo_w.astype(bf16)
    gb3 = gate_b.reshape(E * 2, 1, FBLK)
    vb3 = val_b.reshape(E * 2, 1, FBLK)
    ob3 = wo_b.reshape(E, 1, D)

    ydisp = pl.pallas_call(
        _gmm_kernel,
        grid_spec=pltpu.PrefetchScalarGridSpec(
            num_scalar_prefetch=2,
            grid=(2, NB),
            in_specs=[
                pl.BlockSpec((M, D), lambda f, b, be_r, na_r: (b, 0)),
                pl.BlockSpec((1, FBLK, D), lambda f, b, be_r, na_r: (be_r[b], f, 0)),
                pl.BlockSpec((1, FBLK, D), lambda f, b, be_r, na_r: (be_r[b], f, 0)),
                pl.BlockSpec((1, D, FBLK), lambda f, b, be_r, na_r: (be_r[b], 0, f)),
                pl.BlockSpec((1, 1, FBLK), lambda f, b, be_r, na_r: (be_r[b] * 2 + f, 0, 0)),
                pl.BlockSpec((1, 1, FBLK), lambda f, b, be_r, na_r: (be_r[b] * 2 + f, 0, 0)),
                pl.BlockSpec((1, 1, D), lambda f, b, be_r, na_r: (be_r[b], 0, 0)),
            ],
            out_specs=pl.BlockSpec((1, M, D), lambda f, b, be_r, na_r: (f, b, 0)),
        ),
        out_shape=jax.ShapeDtypeStruct((2, RMAX, D), bf16),
        compiler_params=_cparams(("arbitrary", "arbitrary")),
        interpret=_INTERPRET,
    )(be, nact, xdisp, gwb, vwb, owb, gb3, vb3, ob3)

    ygat = _sc_gather(ydisp.reshape(2 * RMAX, D), gidx)

    out = pl.pallas_call(
        _combine_kernel,
        grid=(NSB,),
        in_specs=[
            pl.BlockSpec((SBLK, D), lambda s: (s, 0)),
            pl.BlockSpec((SBLK, 1), lambda s: (s, 0)),
            pl.BlockSpec((SBLK, 1), lambda s: (s, 0)),
            pl.BlockSpec((SBLK, D), lambda s: (s, 0)),
            pl.BlockSpec((SBLK, D), lambda s: (s + NSB, 0)),
            pl.BlockSpec((SBLK, D), lambda s: (s + 2 * NSB, 0)),
            pl.BlockSpec((SBLK, D), lambda s: (s + 3 * NSB, 0)),
        ],
        out_specs=pl.BlockSpec((SBLK, D), lambda s: (s, 0)),
        out_shape=jax.ShapeDtypeStruct((S, D), f32),
        compiler_params=_cparams(("arbitrary",)),
        interpret=_INTERPRET,
    )(x1, wlo, whi, ygat, ygat, ygat, ygat)

    return out.reshape(B, S, D)


# speedup vs baseline: 1.7181x; 1.7181x over previous
import jax
import jax.numpy as jnp
from jax.experimental import pallas as pl
from jax.experimental.pallas import tpu as pltpu
_INTERPRET = False

B, S, D, H, F, E = 1, 2048, 1024, 16, 4096, 8
HD = D // H
HALF = HD // 2
SBLK = 256
NSB = S // SBLK
QBLK = 512
EPAD = 128
FBLK = F // 2
NEG = -1e30
M = 256
NB = (2 * S) // M + E
RMAX = NB * M
def _ln_f32(xb, g, b):
    m = jnp.mean(xb, axis=-1, keepdims=True)
    v = jnp.mean((xb - m) ** 2, axis=-1, keepdims=True)
    return (xb - m) / jnp.sqrt(v + 1e-5) * g + b


def _qkv_kernel(x_ref, inwT_ref, inb_ref, g_ref, b_ref, inv_ref, pswap_ref,
                q2_ref, k2_ref, v2_ref):
    s = pl.program_id(0)
    xb = x_ref[...]
    h = _ln_f32(xb, g_ref[...], b_ref[...])
    w = inwT_ref[...]
    qkv = jnp.dot(h.astype(jnp.bfloat16), w,
                  preferred_element_type=jnp.float32) + inb_ref[...]
    q, k, v = qkv[:, :D], qkv[:, D:2 * D], qkv[:, 2 * D:]
    pos = (s * SBLK
           + jax.lax.broadcasted_iota(jnp.int32, (SBLK, HALF), 0)).astype(jnp.float32)
    ang = pos * inv_ref[...]
    c32, s32 = jnp.cos(ang), jnp.sin(ang)
    c64 = jnp.concatenate([c32, c32], axis=1)
    s64 = jnp.concatenate([s32, s32], axis=1)
    cf = jnp.concatenate([c64] * H, axis=1)
    sf = jnp.concatenate([s64] * H, axis=1)
    psw = pswap_ref[...]
    qsw = jnp.dot(q.astype(jnp.bfloat16), psw, preferred_element_type=jnp.float32)
    ksw = jnp.dot(k.astype(jnp.bfloat16), psw, preferred_element_type=jnp.float32)
    rq = (q * cf + qsw * sf).astype(jnp.bfloat16)
    rk = (k * cf + ksw * sf).astype(jnp.bfloat16)
    q2_ref[...] = (jnp.dot(rq, w[:, :D], preferred_element_type=jnp.float32)
                   + inb_ref[:, :D]).astype(jnp.bfloat16)
    k2_ref[...] = (jnp.dot(rk, w[:, D:2 * D], preferred_element_type=jnp.float32)
                   + inb_ref[:, D:2 * D]).astype(jnp.bfloat16)
    v2_ref[...] = (jnp.dot(v.astype(jnp.bfloat16), w[:, 2 * D:],
                           preferred_element_type=jnp.float32)
                   + inb_ref[:, 2 * D:]).astype(jnp.bfloat16)


def _attn_kernel(q_ref, k_ref, v_ref, o_ref):
    qb = q_ref[...]
    kb = k_ref[...]
    vb = v_ref[...]
    outs = []
    for hh in range(2):
        q1 = qb[:, hh * HD:(hh + 1) * HD]
        k1 = kb[:, hh * HD:(hh + 1) * HD]
        v1 = vb[:, hh * HD:(hh + 1) * HD]
        sc = jax.lax.dot_general(q1, k1, (((1,), (1,)), ((), ())),
                                 preferred_element_type=jnp.float32) * 0.125
        m = jnp.max(sc, axis=1, keepdims=True)
        p = jnp.exp(sc - m)
        z = jnp.sum(p, axis=1, keepdims=True)
        pb = (p / z).astype(jnp.bfloat16)
        outs.append(jnp.dot(pb, v1, preferred_element_type=jnp.float32))
    o_ref[...] = jnp.concatenate(outs, axis=1).astype(jnp.bfloat16)


def _post_kernel(ctx_ref, x_ref, outwT_ref, outb_ref, g2_ref, b2_ref, rwT_ref,
                 x1_ref, h2_ref, w_ref):
    ctx = ctx_ref[...]
    attn = jnp.dot(ctx, outwT_ref[...],
                   preferred_element_type=jnp.float32) + outb_ref[...]
    x1 = x_ref[...] + attn
    x1_ref[...] = x1
    h2 = _ln_f32(x1, g2_ref[...], b2_ref[...])
    h2_ref[...] = h2.astype(jnp.bfloat16)
    logits = jnp.dot(h2, rwT_ref[...], preferred_element_type=jnp.float32)
    lane = jax.lax.broadcasted_iota(jnp.int32, (SBLK, EPAD), 1)
    l = jnp.where(lane < E, logits, NEG)
    m1 = jnp.max(l, axis=1, keepdims=True)
    i1 = jnp.min(jnp.where(l == m1, lane, EPAD), axis=1, keepdims=True)
    l2 = jnp.where(lane == i1, NEG, l)
    m2 = jnp.max(l2, axis=1, keepdims=True)
    i2 = jnp.min(jnp.where(l2 == m2, lane, EPAD), axis=1, keepdims=True)
    ex = jnp.where(lane < E, jnp.exp(l - m1), 0.0)
    zz = jnp.sum(ex, axis=1, keepdims=True)
    p = ex / zz
    w_ref[...] = jnp.where((lane == i1) | (lane == i2), p, -1.0)


def _route_kernel(w_ref, pk_ref):
    w = w_ref[...]
    lane = jax.lax.broadcasted_iota(jnp.int32, (S, EPAD), 1)
    sub = jax.lax.broadcasted_iota(jnp.int32, (S, EPAD), 0)
    sel = w >= 0.0
    oh = jnp.where(sel, 1.0, 0.0)
    r_i = jax.lax.broadcasted_iota(jnp.int32, (M, M), 0)
    c_i = jax.lax.broadcasted_iota(jnp.int32, (M, M), 1)
    tri = jnp.where(r_i > c_i, 1.0, 0.0).astype(jnp.bfloat16)
    carry = jnp.zeros((1, EPAD), jnp.float32)
    ranks = []
    for blk in range(S // M):
        ob = oh[blk * M:(blk + 1) * M, :]
        rb = jax.lax.dot_general(tri, ob.astype(jnp.bfloat16),
                                 (((1,), (0,)), ((), ())),
                                 preferred_element_type=jnp.float32)
        ranks.append(rb + carry)
        carry = carry + jnp.sum(ob, axis=0, keepdims=True)
    rank = jnp.concatenate(ranks, axis=0)
    counts = carry
    pc = jnp.ceil(counts * (1.0 / M)) * M
    off_sc = []
    run = jnp.float32(0.0)
    for e in range(E):
        off_sc.append(run)
        run = run + pc[0, e]
    nact = run * (1.0 / M)
    off_vec = jnp.zeros((1, EPAD), jnp.float32)
    end_vec = jnp.full((1, EPAD), jnp.float32(NEG * -1.0))
    for e in range(E):
        lm = (lane[:1, :] == e)
        off_vec = jnp.where(lm, off_sc[e], off_vec)
        end_vec = jnp.where(lm, off_sc[e] + pc[0, e], end_vec)
    val = rank + off_vec
    lane_sel = jnp.where(sel, lane, EPAD)
    e_lo = jnp.min(lane_sel, axis=1, keepdims=True)
    lane_sel2 = jnp.where(sel, lane, -1)
    e_hi = jnp.max(lane_sel2, axis=1, keepdims=True)
    d_lo = jnp.sum(jnp.where(lane == e_lo, val, 0.0), axis=1, keepdims=True)
    d_hi = jnp.sum(jnp.where(lane == e_hi, val, 0.0), axis=1, keepdims=True)
    w_lo = jnp.sum(jnp.where(lane == e_lo, jnp.maximum(w, 0.0), 0.0),
                   axis=1, keepdims=True)
    w_hi = jnp.sum(jnp.where(lane == e_hi, jnp.maximum(w, 0.0), 0.0),
                   axis=1, keepdims=True)
    bvals = (sub[:, :1] * M).astype(jnp.float32)
    ge = (bvals >= end_vec) & (lane[:1, :] < E)
    be = jnp.sum(jnp.where(ge, 1.0, 0.0), axis=1, keepdims=True)
    be = jnp.minimum(be, E - 1)
    pk = jnp.where(lane == 0, d_lo, 0.0)
    pk = pk + jnp.where(lane == 1, d_hi, 0.0)
    pk = pk + jnp.where(lane == 2, w_lo, 0.0)
    pk = pk + jnp.where(lane == 3, w_hi, 0.0)
    pk = pk + jnp.where(lane == 4, be, 0.0)
    pk = pk + jnp.where(lane == 5, nact, 0.0)
    pk_ref[...] = pk


def _gmm_kernel(be_ref, na_ref, xd_ref, gw_ref, vw_ref, ow_ref,
                gb_ref, vb_ref, ob_ref, y_ref):
    f = pl.program_id(0)
    b = pl.program_id(1)

    @pl.when(b < na_ref[0])
    def _():
        xb = xd_ref[...]
        g = jax.lax.dot_general(xb, gw_ref[0], (((1,), (1,)), ((), ())),
                                preferred_element_type=jnp.float32) + gb_ref[0, 0]
        v = jax.lax.dot_general(xb, vw_ref[0], (((1,), (1,)), ((), ())),
                                preferred_element_type=jnp.float32) + vb_ref[0, 0]
        hh = (v * (g * jax.nn.sigmoid(g))).astype(jnp.bfloat16)
        o = jax.lax.dot_general(hh, ow_ref[0], (((1,), (1,)), ((), ())),
                                preferred_element_type=jnp.float32)
        ob = ob_ref[0, 0][None, :] * (f == 0).astype(jnp.float32)
        y_ref[0] = (o + ob).astype(jnp.bfloat16)

    @pl.when(b >= na_ref[0])
    def _():
        y_ref[0] = jnp.zeros((M, D), jnp.bfloat16)


def _dispatch_kernel(dl_ref, dh_ref, h2_ref, xd_ref):
    rb = pl.program_id(0)
    riota = (rb * M
             + jax.lax.broadcasted_iota(jnp.int32, (S, M), 1)).astype(jnp.float32)
    dl = dl_ref[...]
    dh = dh_ref[...]
    q = jnp.where((dl == riota) | (dh == riota), 1.0, 0.0).astype(jnp.bfloat16)
    xd_ref[...] = jax.lax.dot_general(
        q, h2_ref[...], (((0,), (0,)), ((), ())),
        preferred_element_type=jnp.float32).astype(jnp.bfloat16)


def _comb_kernel(dl_ref, dh_ref, wlo_ref, whi_ref, x1_ref, y0_ref, y1_ref,
                 out_ref):
    rb = pl.program_id(0)
    riota = (rb * M
             + jax.lax.broadcasted_iota(jnp.int32, (S, M), 1)).astype(jnp.float32)
    dl = dl_ref[...]
    dh = dh_ref[...]
    qw = (jnp.where(dl == riota, 1.0, 0.0) * wlo_ref[...]
          + jnp.where(dh == riota, 1.0, 0.0) * whi_ref[...]).astype(jnp.bfloat16)
    y = y0_ref[0] + y1_ref[0]
    contrib = jax.lax.dot_general(qw, y, (((1,), (0,)), ((), ())),
                                  preferred_element_type=jnp.float32)

    @pl.when(rb == 0)
    def _():
        out_ref[...] = x1_ref[...] + contrib

    @pl.when(rb > 0)
    def _():
        out_ref[...] += contrib


def _cparams(sem):
    return pltpu.CompilerParams(dimension_semantics=sem)


def kernel(x, ln1_g, ln1_b, in_w, in_b, out_w, out_b, ln2_g, ln2_b,
           router_w, gate_w, gate_b, val_w, val_b, wo_w, wo_b):
    f32, bf16 = jnp.float32, jnp.bfloat16
    x2 = x.reshape(S, D)
    inwT = in_w.T.astype(bf16)
    inb = in_b.reshape(1, 3 * D)
    g1 = ln1_g.reshape(1, D)
    b1 = ln1_b.reshape(1, D)
    inv = (1.0 / (10000.0 ** (jnp.arange(HALF, dtype=f32) / HALF))).reshape(1, HALF)
    eye = jnp.eye(HALF, dtype=f32)
    zer = jnp.zeros((HALF, HALF), f32)
    p64 = jnp.concatenate([
        jnp.concatenate([zer, eye], axis=1),
        jnp.concatenate([-eye, zer], axis=1)], axis=0)
    pswap = jnp.kron(jnp.eye(H, dtype=f32), p64).astype(bf16)

    q2, k2, v2 = pl.pallas_call(
        _qkv_kernel,
        grid=(NSB,),
        in_specs=[
            pl.BlockSpec((SBLK, D), lambda s: (s, 0)),
            pl.BlockSpec((D, 3 * D), lambda s: (0, 0)),
            pl.BlockSpec((1, 3 * D), lambda s: (0, 0)),
            pl.BlockSpec((1, D), lambda s: (0, 0)),
            pl.BlockSpec((1, D), lambda s: (0, 0)),
            pl.BlockSpec((1, HALF), lambda s: (0, 0)),
            pl.BlockSpec((D, D), lambda s: (0, 0)),
        ],
        out_specs=[pl.BlockSpec((SBLK, D), lambda s: (s, 0))] * 3,
        out_shape=[jax.ShapeDtypeStruct((S, D), bf16)] * 3,
        compiler_params=_cparams(("arbitrary",)),
        interpret=_INTERPRET,
    )(x2, inwT, inb, g1, b1, inv, pswap)

    ctx = pl.pallas_call(
        _attn_kernel,
        grid=(H // 2, S // QBLK),
        in_specs=[
            pl.BlockSpec((QBLK, 2 * HD), lambda hp, sq: (sq, hp)),
            pl.BlockSpec((S, 2 * HD), lambda hp, sq: (0, hp)),
            pl.BlockSpec((S, 2 * HD), lambda hp, sq: (0, hp)),
        ],
        out_specs=pl.BlockSpec((QBLK, 2 * HD), lambda hp, sq: (sq, hp)),
        out_shape=jax.ShapeDtypeStruct((S, D), bf16),
        compiler_params=_cparams(("arbitrary", "arbitrary")),
        interpret=_INTERPRET,
    )(q2, k2, v2)

    outwT = out_w.T.astype(bf16)
    outb = out_b.reshape(1, D)
    g2 = ln2_g.reshape(1, D)
    b2 = ln2_b.reshape(1, D)
    rwT = jnp.zeros((D, EPAD), f32).at[:, :E].set(router_w.T)

    x1, h2b, w = pl.pallas_call(
        _post_kernel,
        grid=(NSB,),
        in_specs=[
            pl.BlockSpec((SBLK, D), lambda s: (s, 0)),
            pl.BlockSpec((SBLK, D), lambda s: (s, 0)),
            pl.BlockSpec((D, D), lambda s: (0, 0)),
            pl.BlockSpec((1, D), lambda s: (0, 0)),
            pl.BlockSpec((1, D), lambda s: (0, 0)),
            pl.BlockSpec((1, D), lambda s: (0, 0)),
            pl.BlockSpec((D, EPAD), lambda s: (0, 0)),
        ],
        out_specs=[
            pl.BlockSpec((SBLK, D), lambda s: (s, 0)),
            pl.BlockSpec((SBLK, D), lambda s: (s, 0)),
            pl.BlockSpec((SBLK, EPAD), lambda s: (s, 0)),
        ],
        out_shape=[
            jax.ShapeDtypeStruct((S, D), f32),
            jax.ShapeDtypeStruct((S, D), bf16),
            jax.ShapeDtypeStruct((S, EPAD), f32),
        ],
        compiler_params=_cparams(("arbitrary",)),
        interpret=_INTERPRET,
    )(ctx, x2, outwT, outb, g2, b2, rwT)

    pk = pl.pallas_call(
        _route_kernel,
        in_specs=[pl.BlockSpec((S, EPAD), lambda: (0, 0))],
        out_specs=pl.BlockSpec((S, EPAD), lambda: (0, 0)),
        out_shape=jax.ShapeDtypeStruct((S, EPAD), f32),
        interpret=_INTERPRET,
    )(w)

    dl = pk[:, 0:1]
    dh = pk[:, 1:2]
    wlo = pk[:, 2:3]
    whi = pk[:, 3:4]
    be = pk[:NB, 4].astype(jnp.int32)
    nact = pk[0:1, 5].astype(jnp.int32)

    xdisp = pl.pallas_call(
        _dispatch_kernel,
        grid=(NB,),
        in_specs=[
            pl.BlockSpec((S, 1), lambda rb: (0, 0)),
            pl.BlockSpec((S, 1), lambda rb: (0, 0)),
            pl.BlockSpec((S, D), lambda rb: (0, 0)),
        ],
        out_specs=pl.BlockSpec((M, D), lambda rb: (rb, 0)),
        out_shape=jax.ShapeDtypeStruct((RMAX, D), bf16),
        compiler_params=_cparams(("arbitrary",)),
        interpret=_INTERPRET,
    )(dl, dh, h2b)

    gwb = gate_w.astype(bf16)
    vwb = val_w.astype(bf16)
    owb = wo_w.astype(bf16)
    gb3 = gate_b.reshape(E * 2, 1, FBLK)
    vb3 = val_b.reshape(E * 2, 1, FBLK)
    ob3 = wo_b.reshape(E, 1, D)

    ydisp = pl.pallas_call(
        _gmm_kernel,
        grid_spec=pltpu.PrefetchScalarGridSpec(
            num_scalar_prefetch=2,
            grid=(2, NB),
            in_specs=[
                pl.BlockSpec((M, D), lambda f, b, be_r, na_r: (b, 0)),
                pl.BlockSpec((1, FBLK, D), lambda f, b, be_r, na_r: (be_r[b], f, 0)),
                pl.BlockSpec((1, FBLK, D), lambda f, b, be_r, na_r: (be_r[b], f, 0)),
                pl.BlockSpec((1, D, FBLK), lambda f, b, be_r, na_r: (be_r[b], 0, f)),
                pl.BlockSpec((1, 1, FBLK), lambda f, b, be_r, na_r: (be_r[b] * 2 + f, 0, 0)),
                pl.BlockSpec((1, 1, FBLK), lambda f, b, be_r, na_r: (be_r[b] * 2 + f, 0, 0)),
                pl.BlockSpec((1, 1, D), lambda f, b, be_r, na_r: (be_r[b], 0, 0)),
            ],
            out_specs=pl.BlockSpec((1, M, D), lambda f, b, be_r, na_r: (f, b, 0)),
        ),
        out_shape=jax.ShapeDtypeStruct((2, RMAX, D), bf16),
        compiler_params=_cparams(("arbitrary", "arbitrary")),
        interpret=_INTERPRET,
    )(be, nact, xdisp, gwb, vwb, owb, gb3, vb3, ob3)

    out = pl.pallas_call(
        _comb_kernel,
        grid=(NB,),
        in_specs=[
            pl.BlockSpec((S, 1), lambda rb: (0, 0)),
            pl.BlockSpec((S, 1), lambda rb: (0, 0)),
            pl.BlockSpec((S, 1), lambda rb: (0, 0)),
            pl.BlockSpec((S, 1), lambda rb: (0, 0)),
            pl.BlockSpec((S, D), lambda rb: (0, 0)),
            pl.BlockSpec((1, M, D), lambda rb: (0, rb, 0)),
            pl.BlockSpec((1, M, D), lambda rb: (1, rb, 0)),
        ],
        out_specs=pl.BlockSpec((S, D), lambda rb: (0, 0)),
        out_shape=jax.ShapeDtypeStruct((S, D), f32),
        compiler_params=_cparams(("arbitrary",)),
        interpret=_INTERPRET,
    )(dl, dh, wlo, whi, x1, ydisp, ydisp)

    return out.reshape(B, S, D)


# speedup vs baseline: 1.8335x; 1.0672x over previous
import jax
import jax.numpy as jnp
from jax.experimental import pallas as pl
from jax.experimental.pallas import tpu as pltpu
_INTERPRET = False

B, S, D, H, F, E = 1, 2048, 1024, 16, 4096, 8
HD = D // H
HALF = HD // 2
SBLK = 256
NSB = S // SBLK
QBLK = 512
EPAD = 128
FBLK = F // 2
NEG = -1e30
M = 256
NB = (2 * S) // M + E
RMAX = NB * M
def _ln_f32(xb, g, b):
    m = jnp.mean(xb, axis=-1, keepdims=True)
    v = jnp.mean((xb - m) ** 2, axis=-1, keepdims=True)
    return (xb - m) / jnp.sqrt(v + 1e-5) * g + b


def _qkv_kernel(x_ref, inwT_ref, inb_ref, g_ref, b_ref, inv_ref, pswap_ref,
                q2_ref, k2_ref, v2_ref):
    s = pl.program_id(0)
    xb = x_ref[...]
    h = _ln_f32(xb, g_ref[...], b_ref[...])
    w = inwT_ref[...]
    qkv = jnp.dot(h.astype(jnp.bfloat16), w,
                  preferred_element_type=jnp.float32) + inb_ref[...]
    q, k, v = qkv[:, :D], qkv[:, D:2 * D], qkv[:, 2 * D:]
    pos = (s * SBLK
           + jax.lax.broadcasted_iota(jnp.int32, (SBLK, HALF), 0)).astype(jnp.float32)
    ang = pos * inv_ref[...]
    c32, s32 = jnp.cos(ang), jnp.sin(ang)
    c64 = jnp.concatenate([c32, c32], axis=1)
    s64 = jnp.concatenate([s32, s32], axis=1)
    cf = jnp.concatenate([c64] * H, axis=1)
    sf = jnp.concatenate([s64] * H, axis=1)
    psw = pswap_ref[...]
    qsw = jnp.dot(q.astype(jnp.bfloat16), psw, preferred_element_type=jnp.float32)
    ksw = jnp.dot(k.astype(jnp.bfloat16), psw, preferred_element_type=jnp.float32)
    rq = (q * cf + qsw * sf).astype(jnp.bfloat16)
    rk = (k * cf + ksw * sf).astype(jnp.bfloat16)
    q2_ref[...] = (jnp.dot(rq, w[:, :D], preferred_element_type=jnp.float32)
                   + inb_ref[:, :D]).astype(jnp.bfloat16)
    k2_ref[...] = (jnp.dot(rk, w[:, D:2 * D], preferred_element_type=jnp.float32)
                   + inb_ref[:, D:2 * D]).astype(jnp.bfloat16)
    v2_ref[...] = (jnp.dot(v.astype(jnp.bfloat16), w[:, 2 * D:],
                           preferred_element_type=jnp.float32)
                   + inb_ref[:, 2 * D:]).astype(jnp.bfloat16)


def _attn_kernel(q_ref, k_ref, v_ref, o_ref):
    qb = q_ref[...]
    kb = k_ref[...]
    vb = v_ref[...]
    outs = []
    for hh in range(2):
        q1 = qb[:, hh * HD:(hh + 1) * HD]
        k1 = kb[:, hh * HD:(hh + 1) * HD]
        v1 = vb[:, hh * HD:(hh + 1) * HD]
        sc = jax.lax.dot_general(q1, k1, (((1,), (1,)), ((), ())),
                                 preferred_element_type=jnp.float32) * 0.125
        m = jnp.max(sc, axis=1, keepdims=True)
        pb = jnp.exp((sc - m).astype(jnp.bfloat16))
        z = jnp.sum(pb, axis=1, keepdims=True, dtype=jnp.float32)
        ctx = jnp.dot(pb, v1, preferred_element_type=jnp.float32)
        outs.append(ctx / z)
    o_ref[...] = jnp.concatenate(outs, axis=1).astype(jnp.bfloat16)


def _post_kernel(ctx_ref, x_ref, outwT_ref, outb_ref, g2_ref, b2_ref, rwT_ref,
                 x1_ref, h2_ref, w_ref):
    ctx = ctx_ref[...]
    attn = jnp.dot(ctx, outwT_ref[...],
                   preferred_element_type=jnp.float32) + outb_ref[...]
    x1 = x_ref[...] + attn
    x1_ref[...] = x1
    h2 = _ln_f32(x1, g2_ref[...], b2_ref[...])
    h2_ref[...] = h2.astype(jnp.bfloat16)
    logits = jnp.dot(h2, rwT_ref[...], preferred_element_type=jnp.float32)
    lane = jax.lax.broadcasted_iota(jnp.int32, (SBLK, EPAD), 1)
    l = jnp.where(lane < E, logits, NEG)
    m1 = jnp.max(l, axis=1, keepdims=True)
    i1 = jnp.min(jnp.where(l == m1, lane, EPAD), axis=1, keepdims=True)
    l2 = jnp.where(lane == i1, NEG, l)
    m2 = jnp.max(l2, axis=1, keepdims=True)
    i2 = jnp.min(jnp.where(l2 == m2, lane, EPAD), axis=1, keepdims=True)
    ex = jnp.where(lane < E, jnp.exp(l - m1), 0.0)
    zz = jnp.sum(ex, axis=1, keepdims=True)
    p = ex / zz
    w_ref[...] = jnp.where((lane == i1) | (lane == i2), p, -1.0)


def _route_kernel(w_ref, pk_ref):
    w = w_ref[...]
    lane = jax.lax.broadcasted_iota(jnp.int32, (S, EPAD), 1)
    sub = jax.lax.broadcasted_iota(jnp.int32, (S, EPAD), 0)
    sel = w >= 0.0
    oh = jnp.where(sel, 1.0, 0.0)
    r_i = jax.lax.broadcasted_iota(jnp.int32, (M, M), 0)
    c_i = jax.lax.broadcasted_iota(jnp.int32, (M, M), 1)
    tri = jnp.where(r_i > c_i, 1.0, 0.0).astype(jnp.bfloat16)
    carry = jnp.zeros((1, EPAD), jnp.float32)
    ranks = []
    for blk in range(S // M):
        ob = oh[blk * M:(blk + 1) * M, :]
        rb = jax.lax.dot_general(tri, ob.astype(jnp.bfloat16),
                                 (((1,), (0,)), ((), ())),
                                 preferred_element_type=jnp.float32)
        ranks.append(rb + carry)
        carry = carry + jnp.sum(ob, axis=0, keepdims=True)
    rank = jnp.concatenate(ranks, axis=0)
    counts = carry
    pc = jnp.ceil(counts * (1.0 / M)) * M
    off_sc = []
    run = jnp.float32(0.0)
    for e in range(E):
        off_sc.append(run)
        run = run + pc[0, e]
    nact = run * (1.0 / M)
    off_vec = jnp.zeros((1, EPAD), jnp.float32)
    end_vec = jnp.full((1, EPAD), jnp.float32(NEG * -1.0))
    for e in range(E):
        lm = (lane[:1, :] == e)
        off_vec = jnp.where(lm, off_sc[e], off_vec)
        end_vec = jnp.where(lm, off_sc[e] + pc[0, e], end_vec)
    val = rank + off_vec
    lane_sel = jnp.where(sel, lane, EPAD)
    e_lo = jnp.min(lane_sel, axis=1, keepdims=True)
    lane_sel2 = jnp.where(sel, lane, -1)
    e_hi = jnp.max(lane_sel2, axis=1, keepdims=True)
    d_lo = jnp.sum(jnp.where(lane == e_lo, val, 0.0), axis=1, keepdims=True)
    d_hi = jnp.sum(jnp.where(lane == e_hi, val, 0.0), axis=1, keepdims=True)
    w_lo = jnp.sum(jnp.where(lane == e_lo, jnp.maximum(w, 0.0), 0.0),
                   axis=1, keepdims=True)
    w_hi = jnp.sum(jnp.where(lane == e_hi, jnp.maximum(w, 0.0), 0.0),
                   axis=1, keepdims=True)
    bvals = (sub[:, :1] * M).astype(jnp.float32)
    ge = (bvals >= end_vec) & (lane[:1, :] < E)
    be = jnp.sum(jnp.where(ge, 1.0, 0.0), axis=1, keepdims=True)
    be = jnp.minimum(be, E - 1)
    pk = jnp.where(lane == 0, d_lo, 0.0)
    pk = pk + jnp.where(lane == 1, d_hi, 0.0)
    pk = pk + jnp.where(lane == 2, w_lo, 0.0)
    pk = pk + jnp.where(lane == 3, w_hi, 0.0)
    pk = pk + jnp.where(lane == 4, be, 0.0)
    pk = pk + jnp.where(lane == 5, nact, 0.0)
    pk_ref[...] = pk


def _gmm_kernel(be_ref, na_ref, xd_ref, gw_ref, vw_ref, ow_ref,
                gb_ref, vb_ref, ob_ref, y_ref):
    f = pl.program_id(0)
    b = pl.program_id(1)

    @pl.when(b < na_ref[0])
    def _():
        xb = xd_ref[...]
        g = jax.lax.dot_general(xb, gw_ref[0], (((1,), (1,)), ((), ())),
                                preferred_element_type=jnp.float32) + gb_ref[0, 0]
        v = jax.lax.dot_general(xb, vw_ref[0], (((1,), (1,)), ((), ())),
                                preferred_element_type=jnp.float32) + vb_ref[0, 0]
        hh = (v * (g * jax.nn.sigmoid(g))).astype(jnp.bfloat16)
        o = jax.lax.dot_general(hh, ow_ref[0], (((1,), (1,)), ((), ())),
                                preferred_element_type=jnp.float32)
        ob = ob_ref[0, 0][None, :] * (f == 0).astype(jnp.float32)
        y_ref[0] = (o + ob).astype(jnp.bfloat16)

    @pl.when(b >= na_ref[0])
    def _():
        y_ref[0] = jnp.zeros((M, D), jnp.bfloat16)


def _dispatch_kernel(dl_ref, dh_ref, h2_ref, xd_ref):
    rb = pl.program_id(0)
    riota = (rb * M
             + jax.lax.broadcasted_iota(jnp.int32, (S, M), 1)).astype(jnp.float32)
    dl = dl_ref[...]
    dh = dh_ref[...]
    q = jnp.where((dl == riota) | (dh == riota), 1.0, 0.0).astype(jnp.bfloat16)
    xd_ref[...] = jax.lax.dot_general(
        q, h2_ref[...], (((0,), (0,)), ((), ())),
        preferred_element_type=jnp.float32).astype(jnp.bfloat16)


def _comb_kernel(na_ref, dl_ref, dh_ref, wlo_ref, whi_ref, x1_ref, y0_ref,
                 y1_ref, out_ref):
    rb = pl.program_id(0)

    @pl.when(rb == 0)
    def _():
        out_ref[...] = x1_ref[...]

    @pl.when(rb < na_ref[0])
    def _():
        riota = (rb * M
                 + jax.lax.broadcasted_iota(jnp.int32, (S, M), 1)).astype(jnp.float32)
        dl = dl_ref[...]
        dh = dh_ref[...]
        qw = (jnp.where(dl == riota, 1.0, 0.0) * wlo_ref[...]
              + jnp.where(dh == riota, 1.0, 0.0) * whi_ref[...]).astype(jnp.bfloat16)
        y = y0_ref[0] + y1_ref[0]
        out_ref[...] += jax.lax.dot_general(qw, y, (((1,), (0,)), ((), ())),
                                            preferred_element_type=jnp.float32)


def _cparams(sem):
    return pltpu.CompilerParams(dimension_semantics=sem)


def kernel(x, ln1_g, ln1_b, in_w, in_b, out_w, out_b, ln2_g, ln2_b,
           router_w, gate_w, gate_b, val_w, val_b, wo_w, wo_b):
    f32, bf16 = jnp.float32, jnp.bfloat16
    x2 = x.reshape(S, D)
    inwT = in_w.T.astype(bf16)
    inb = in_b.reshape(1, 3 * D)
    g1 = ln1_g.reshape(1, D)
    b1 = ln1_b.reshape(1, D)
    inv = (1.0 / (10000.0 ** (jnp.arange(HALF, dtype=f32) / HALF))).reshape(1, HALF)
    eye = jnp.eye(HALF, dtype=f32)
    zer = jnp.zeros((HALF, HALF), f32)
    p64 = jnp.concatenate([
        jnp.concatenate([zer, eye], axis=1),
        jnp.concatenate([-eye, zer], axis=1)], axis=0)
    pswap = jnp.kron(jnp.eye(H, dtype=f32), p64).astype(bf16)

    q2, k2, v2 = pl.pallas_call(
        _qkv_kernel,
        grid=(NSB,),
        in_specs=[
            pl.BlockSpec((SBLK, D), lambda s: (s, 0)),
            pl.BlockSpec((D, 3 * D), lambda s: (0, 0)),
            pl.BlockSpec((1, 3 * D), lambda s: (0, 0)),
            pl.BlockSpec((1, D), lambda s: (0, 0)),
            pl.BlockSpec((1, D), lambda s: (0, 0)),
            pl.BlockSpec((1, HALF), lambda s: (0, 0)),
            pl.BlockSpec((D, D), lambda s: (0, 0)),
        ],
        out_specs=[pl.BlockSpec((SBLK, D), lambda s: (s, 0))] * 3,
        out_shape=[jax.ShapeDtypeStruct((S, D), bf16)] * 3,
        compiler_params=_cparams(("arbitrary",)),
        interpret=_INTERPRET,
    )(x2, inwT, inb, g1, b1, inv, pswap)

    ctx = pl.pallas_call(
        _attn_kernel,
        grid=(H // 2, S // QBLK),
        in_specs=[
            pl.BlockSpec((QBLK, 2 * HD), lambda hp, sq: (sq, hp)),
            pl.BlockSpec((S, 2 * HD), lambda hp, sq: (0, hp)),
            pl.BlockSpec((S, 2 * HD), lambda hp, sq: (0, hp)),
        ],
        out_specs=pl.BlockSpec((QBLK, 2 * HD), lambda hp, sq: (sq, hp)),
        out_shape=jax.ShapeDtypeStruct((S, D), bf16),
        compiler_params=_cparams(("arbitrary", "arbitrary")),
        interpret=_INTERPRET,
    )(q2, k2, v2)

    outwT = out_w.T.astype(bf16)
    outb = out_b.reshape(1, D)
    g2 = ln2_g.reshape(1, D)
    b2 = ln2_b.reshape(1, D)
    rwT = jnp.zeros((D, EPAD), f32).at[:, :E].set(router_w.T)

    x1, h2b, w = pl.pallas_call(
        _post_kernel,
        grid=(NSB,),
        in_specs=[
            pl.BlockSpec((SBLK, D), lambda s: (s, 0)),
            pl.BlockSpec((SBLK, D), lambda s: (s, 0)),
            pl.BlockSpec((D, D), lambda s: (0, 0)),
            pl.BlockSpec((1, D), lambda s: (0, 0)),
            pl.BlockSpec((1, D), lambda s: (0, 0)),
            pl.BlockSpec((1, D), lambda s: (0, 0)),
            pl.BlockSpec((D, EPAD), lambda s: (0, 0)),
        ],
        out_specs=[
            pl.BlockSpec((SBLK, D), lambda s: (s, 0)),
            pl.BlockSpec((SBLK, D), lambda s: (s, 0)),
            pl.BlockSpec((SBLK, EPAD), lambda s: (s, 0)),
        ],
        out_shape=[
            jax.ShapeDtypeStruct((S, D), f32),
            jax.ShapeDtypeStruct((S, D), bf16),
            jax.ShapeDtypeStruct((S, EPAD), f32),
        ],
        compiler_params=_cparams(("arbitrary",)),
        interpret=_INTERPRET,
    )(ctx, x2, outwT, outb, g2, b2, rwT)

    pk = pl.pallas_call(
        _route_kernel,
        in_specs=[pl.BlockSpec((S, EPAD), lambda: (0, 0))],
        out_specs=pl.BlockSpec((S, EPAD), lambda: (0, 0)),
        out_shape=jax.ShapeDtypeStruct((S, EPAD), f32),
        interpret=_INTERPRET,
    )(w)

    dl = pk[:, 0:1]
    dh = pk[:, 1:2]
    wlo = pk[:, 2:3]
    whi = pk[:, 3:4]
    be = pk[:NB, 4].astype(jnp.int32)
    nact = pk[0:1, 5].astype(jnp.int32)

    xdisp = pl.pallas_call(
        _dispatch_kernel,
        grid=(NB,),
        in_specs=[
            pl.BlockSpec((S, 1), lambda rb: (0, 0)),
            pl.BlockSpec((S, 1), lambda rb: (0, 0)),
            pl.BlockSpec((S, D), lambda rb: (0, 0)),
        ],
        out_specs=pl.BlockSpec((M, D), lambda rb: (rb, 0)),
        out_shape=jax.ShapeDtypeStruct((RMAX, D), bf16),
        compiler_params=_cparams(("arbitrary",)),
        interpret=_INTERPRET,
    )(dl, dh, h2b)

    gwb = gate_w.astype(bf16)
    vwb = val_w.astype(bf16)
    owb = wo_w.astype(bf16)
    gb3 = gate_b.reshape(E * 2, 1, FBLK)
    vb3 = val_b.reshape(E * 2, 1, FBLK)
    ob3 = wo_b.reshape(E, 1, D)

    ydisp = pl.pallas_call(
        _gmm_kernel,
        grid_spec=pltpu.PrefetchScalarGridSpec(
            num_scalar_prefetch=2,
            grid=(2, NB),
            in_specs=[
                pl.BlockSpec((M, D), lambda f, b, be_r, na_r: (b, 0)),
                pl.BlockSpec((1, FBLK, D), lambda f, b, be_r, na_r: (be_r[b], f, 0)),
                pl.BlockSpec((1, FBLK, D), lambda f, b, be_r, na_r: (be_r[b], f, 0)),
                pl.BlockSpec((1, D, FBLK), lambda f, b, be_r, na_r: (be_r[b], 0, f)),
                pl.BlockSpec((1, 1, FBLK), lambda f, b, be_r, na_r: (be_r[b] * 2 + f, 0, 0)),
                pl.BlockSpec((1, 1, FBLK), lambda f, b, be_r, na_r: (be_r[b] * 2 + f, 0, 0)),
                pl.BlockSpec((1, 1, D), lambda f, b, be_r, na_r: (be_r[b], 0, 0)),
            ],
            out_specs=pl.BlockSpec((1, M, D), lambda f, b, be_r, na_r: (f, b, 0)),
        ),
        out_shape=jax.ShapeDtypeStruct((2, RMAX, D), bf16),
        compiler_params=_cparams(("arbitrary", "arbitrary")),
        interpret=_INTERPRET,
    )(be, nact, xdisp, gwb, vwb, owb, gb3, vb3, ob3)

    out = pl.pallas_call(
        _comb_kernel,
        grid_spec=pltpu.PrefetchScalarGridSpec(
            num_scalar_prefetch=1,
            grid=(NB,),
            in_specs=[
                pl.BlockSpec((S, 1), lambda rb, na_r: (0, 0)),
                pl.BlockSpec((S, 1), lambda rb, na_r: (0, 0)),
                pl.BlockSpec((S, 1), lambda rb, na_r: (0, 0)),
                pl.BlockSpec((S, 1), lambda rb, na_r: (0, 0)),
                pl.BlockSpec((S, D), lambda rb, na_r: (0, 0)),
                pl.BlockSpec((1, M, D), lambda rb, na_r: (0, rb, 0)),
                pl.BlockSpec((1, M, D), lambda rb, na_r: (1, rb, 0)),
            ],
            out_specs=pl.BlockSpec((S, D), lambda rb, na_r: (0, 0)),
        ),
        out_shape=jax.ShapeDtypeStruct((S, D), f32),
        compiler_params=_cparams(("arbitrary",)),
        interpret=_INTERPRET,
    )(nact, dl, dh, wlo, whi, x1, ydisp, ydisp)

    return out.reshape(B, S, D)


# speedup vs baseline: 1.8434x; 1.0054x over previous
import jax
import jax.numpy as jnp
from jax.experimental import pallas as pl
from jax.experimental.pallas import tpu as pltpu
_INTERPRET = False

B, S, D, H, F, E = 1, 2048, 1024, 16, 4096, 8
HD = D // H
HALF = HD // 2
SBLK = 256
NSB = S // SBLK
QBLK = 1024
EPAD = 128
FBLK = F // 2
NEG = -1e30
M = 256
NB = (2 * S) // M + E
RMAX = NB * M
def _ln_f32(xb, g, b):
    m = jnp.mean(xb, axis=-1, keepdims=True)
    v = jnp.mean((xb - m) ** 2, axis=-1, keepdims=True)
    return (xb - m) / jnp.sqrt(v + 1e-5) * g + b


def _qkv_kernel(x_ref, inwT_ref, inb_ref, g_ref, b_ref, inv_ref, pswap_ref,
                q2_ref, k2_ref, v2_ref):
    s = pl.program_id(0)
    xb = x_ref[...]
    h = _ln_f32(xb, g_ref[...], b_ref[...])
    w = inwT_ref[...]
    qkv = jnp.dot(h.astype(jnp.bfloat16), w,
                  preferred_element_type=jnp.float32) + inb_ref[...]
    q, k, v = qkv[:, :D], qkv[:, D:2 * D], qkv[:, 2 * D:]
    pos = (s * SBLK
           + jax.lax.broadcasted_iota(jnp.int32, (SBLK, HALF), 0)).astype(jnp.float32)
    ang = pos * inv_ref[...]
    c32, s32 = jnp.cos(ang), jnp.sin(ang)
    c64 = jnp.concatenate([c32, c32], axis=1)
    s64 = jnp.concatenate([s32, s32], axis=1)
    cf = jnp.concatenate([c64] * H, axis=1)
    sf = jnp.concatenate([s64] * H, axis=1)
    psw = pswap_ref[...]
    qsw = jnp.dot(q.astype(jnp.bfloat16), psw, preferred_element_type=jnp.float32)
    ksw = jnp.dot(k.astype(jnp.bfloat16), psw, preferred_element_type=jnp.float32)
    rq = (q * cf + qsw * sf).astype(jnp.bfloat16)
    rk = (k * cf + ksw * sf).astype(jnp.bfloat16)
    q2_ref[...] = (jnp.dot(rq, w[:, :D], preferred_element_type=jnp.float32)
                   + inb_ref[:, :D]).astype(jnp.bfloat16)
    k2_ref[...] = (jnp.dot(rk, w[:, D:2 * D], preferred_element_type=jnp.float32)
                   + inb_ref[:, D:2 * D]).astype(jnp.bfloat16)
    v2_ref[...] = (jnp.dot(v.astype(jnp.bfloat16), w[:, 2 * D:],
                           preferred_element_type=jnp.float32)
                   + inb_ref[:, 2 * D:]).astype(jnp.bfloat16)


def _attn_kernel(q_ref, k_ref, v_ref, o_ref):
    qb = q_ref[...]
    kb = k_ref[...]
    vb = v_ref[...]
    outs = []
    for hh in range(2):
        q1 = qb[:, hh * HD:(hh + 1) * HD]
        k1 = kb[:, hh * HD:(hh + 1) * HD]
        v1 = vb[:, hh * HD:(hh + 1) * HD]
        sc = jax.lax.dot_general(q1, k1, (((1,), (1,)), ((), ())),
                                 preferred_element_type=jnp.float32) * 0.125
        m = jnp.max(sc, axis=1, keepdims=True)
        pb = jnp.exp((sc - m).astype(jnp.bfloat16))
        z = jnp.sum(pb, axis=1, keepdims=True, dtype=jnp.float32)
        ctx = jnp.dot(pb, v1, preferred_element_type=jnp.float32)
        outs.append(ctx / z)
    o_ref[...] = jnp.concatenate(outs, axis=1).astype(jnp.bfloat16)


def _post_kernel(ctx_ref, x_ref, outwT_ref, outb_ref, g2_ref, b2_ref, rwT_ref,
                 x1_ref, h2_ref, w_ref):
    ctx = ctx_ref[...]
    attn = jnp.dot(ctx, outwT_ref[...],
                   preferred_element_type=jnp.float32) + outb_ref[...]
    x1 = x_ref[...] + attn
    x1_ref[...] = x1
    h2 = _ln_f32(x1, g2_ref[...], b2_ref[...])
    h2_ref[...] = h2.astype(jnp.bfloat16)
    logits = jnp.dot(h2, rwT_ref[...], preferred_element_type=jnp.float32)
    lane = jax.lax.broadcasted_iota(jnp.int32, (SBLK, EPAD), 1)
    l = jnp.where(lane < E, logits, NEG)
    m1 = jnp.max(l, axis=1, keepdims=True)
    i1 = jnp.min(jnp.where(l == m1, lane, EPAD), axis=1, keepdims=True)
    l2 = jnp.where(lane == i1, NEG, l)
    m2 = jnp.max(l2, axis=1, keepdims=True)
    i2 = jnp.min(jnp.where(l2 == m2, lane, EPAD), axis=1, keepdims=True)
    ex = jnp.where(lane < E, jnp.exp(l - m1), 0.0)
    zz = jnp.sum(ex, axis=1, keepdims=True)
    p = ex / zz
    w_ref[...] = jnp.where((lane == i1) | (lane == i2), p, -1.0)


def _route_kernel(w_ref, pk_ref):
    w = w_ref[...]
    lane = jax.lax.broadcasted_iota(jnp.int32, (S, EPAD), 1)
    sub = jax.lax.broadcasted_iota(jnp.int32, (S, EPAD), 0)
    sel = w >= 0.0
    oh = jnp.where(sel, 1.0, 0.0)
    r_i = jax.lax.broadcasted_iota(jnp.int32, (M, M), 0)
    c_i = jax.lax.broadcasted_iota(jnp.int32, (M, M), 1)
    tri = jnp.where(r_i > c_i, 1.0, 0.0).astype(jnp.bfloat16)
    carry = jnp.zeros((1, EPAD), jnp.float32)
    ranks = []
    for blk in range(S // M):
        ob = oh[blk * M:(blk + 1) * M, :]
        rb = jax.lax.dot_general(tri, ob.astype(jnp.bfloat16),
                                 (((1,), (0,)), ((), ())),
                                 preferred_element_type=jnp.float32)
        ranks.append(rb + carry)
        carry = carry + jnp.sum(ob, axis=0, keepdims=True)
    rank = jnp.concatenate(ranks, axis=0)
    counts = carry
    pc = jnp.ceil(counts * (1.0 / M)) * M
    off_sc = []
    run = jnp.float32(0.0)
    for e in range(E):
        off_sc.append(run)
        run = run + pc[0, e]
    nact = run * (1.0 / M)
    off_vec = jnp.zeros((1, EPAD), jnp.float32)
    end_vec = jnp.full((1, EPAD), jnp.float32(NEG * -1.0))
    for e in range(E):
        lm = (lane[:1, :] == e)
        off_vec = jnp.where(lm, off_sc[e], off_vec)
        end_vec = jnp.where(lm, off_sc[e] + pc[0, e], end_vec)
    val = rank + off_vec
    lane_sel = jnp.where(sel, lane, EPAD)
    e_lo = jnp.min(lane_sel, axis=1, keepdims=True)
    lane_sel2 = jnp.where(sel, lane, -1)
    e_hi = jnp.max(lane_sel2, axis=1, keepdims=True)
    d_lo = jnp.sum(jnp.where(lane == e_lo, val, 0.0), axis=1, keepdims=True)
    d_hi = jnp.sum(jnp.where(lane == e_hi, val, 0.0), axis=1, keepdims=True)
    w_lo = jnp.sum(jnp.where(lane == e_lo, jnp.maximum(w, 0.0), 0.0),
                   axis=1, keepdims=True)
    w_hi = jnp.sum(jnp.where(lane == e_hi, jnp.maximum(w, 0.0), 0.0),
                   axis=1, keepdims=True)
    bvals = (sub[:, :1] * M).astype(jnp.float32)
    ge = (bvals >= end_vec) & (lane[:1, :] < E)
    be = jnp.sum(jnp.where(ge, 1.0, 0.0), axis=1, keepdims=True)
    be = jnp.minimum(be, E - 1)
    pk = jnp.where(lane == 0, d_lo, 0.0)
    pk = pk + jnp.where(lane == 1, d_hi, 0.0)
    pk = pk + jnp.where(lane == 2, w_lo, 0.0)
    pk = pk + jnp.where(lane == 3, w_hi, 0.0)
    pk = pk + jnp.where(lane == 4, be, 0.0)
    pk = pk + jnp.where(lane == 5, nact, 0.0)
    pk_ref[...] = pk


def _gmm_kernel(be_ref, na_ref, xd_ref, gw_ref, vw_ref, ow_ref,
                gb_ref, vb_ref, ob_ref, y_ref):
    f = pl.program_id(0)
    b = pl.program_id(1)

    @pl.when(b < na_ref[0])
    def _():
        xb = xd_ref[...]
        g = jax.lax.dot_general(xb, gw_ref[0], (((1,), (1,)), ((), ())),
                                preferred_element_type=jnp.float32) + gb_ref[0, 0]
        v = jax.lax.dot_general(xb, vw_ref[0], (((1,), (1,)), ((), ())),
                                preferred_element_type=jnp.float32) + vb_ref[0, 0]
        hh = (v * (g * jax.nn.sigmoid(g))).astype(jnp.bfloat16)
        o = jax.lax.dot_general(hh, ow_ref[0], (((1,), (1,)), ((), ())),
                                preferred_element_type=jnp.float32)
        ob = ob_ref[0, 0][None, :] * (f == 0).astype(jnp.float32)
        y_ref[0] = (o + ob).astype(jnp.bfloat16)

    @pl.when(b >= na_ref[0])
    def _():
        y_ref[0] = jnp.zeros((M, D), jnp.bfloat16)


def _dispatch_kernel(dl_ref, dh_ref, h2_ref, xd_ref):
    rb = pl.program_id(0)
    riota = (rb * M
             + jax.lax.broadcasted_iota(jnp.int32, (S, M), 1)).astype(jnp.float32)
    dl = dl_ref[...]
    dh = dh_ref[...]
    q = jnp.where((dl == riota) | (dh == riota), 1.0, 0.0).astype(jnp.bfloat16)
    xd_ref[...] = jax.lax.dot_general(
        q, h2_ref[...], (((0,), (0,)), ((), ())),
        preferred_element_type=jnp.float32).astype(jnp.bfloat16)


def _comb_kernel(na_ref, dl_ref, dh_ref, wlo_ref, whi_ref, x1_ref, y0_ref,
                 y1_ref, out_ref):
    rb = pl.program_id(0)

    @pl.when(rb == 0)
    def _():
        out_ref[...] = x1_ref[...]

    @pl.when(rb < na_ref[0])
    def _():
        riota = (rb * M
                 + jax.lax.broadcasted_iota(jnp.int32, (S, M), 1)).astype(jnp.float32)
        dl = dl_ref[...]
        dh = dh_ref[...]
        qw = (jnp.where(dl == riota, 1.0, 0.0) * wlo_ref[...]
              + jnp.where(dh == riota, 1.0, 0.0) * whi_ref[...]).astype(jnp.bfloat16)
        y = y0_ref[0] + y1_ref[0]
        out_ref[...] += jax.lax.dot_general(qw, y, (((1,), (0,)), ((), ())),
                                            preferred_element_type=jnp.float32)


def _cparams(sem):
    return pltpu.CompilerParams(dimension_semantics=sem)


def kernel(x, ln1_g, ln1_b, in_w, in_b, out_w, out_b, ln2_g, ln2_b,
           router_w, gate_w, gate_b, val_w, val_b, wo_w, wo_b):
    f32, bf16 = jnp.float32, jnp.bfloat16
    x2 = x.reshape(S, D)
    inwT = in_w.T.astype(bf16)
    inb = in_b.reshape(1, 3 * D)
    g1 = ln1_g.reshape(1, D)
    b1 = ln1_b.reshape(1, D)
    inv = (1.0 / (10000.0 ** (jnp.arange(HALF, dtype=f32) / HALF))).reshape(1, HALF)
    eye = jnp.eye(HALF, dtype=f32)
    zer = jnp.zeros((HALF, HALF), f32)
    p64 = jnp.concatenate([
        jnp.concatenate([zer, eye], axis=1),
        jnp.concatenate([-eye, zer], axis=1)], axis=0)
    pswap = jnp.kron(jnp.eye(H, dtype=f32), p64).astype(bf16)

    q2, k2, v2 = pl.pallas_call(
        _qkv_kernel,
        grid=(NSB,),
        in_specs=[
            pl.BlockSpec((SBLK, D), lambda s: (s, 0)),
            pl.BlockSpec((D, 3 * D), lambda s: (0, 0)),
            pl.BlockSpec((1, 3 * D), lambda s: (0, 0)),
            pl.BlockSpec((1, D), lambda s: (0, 0)),
            pl.BlockSpec((1, D), lambda s: (0, 0)),
            pl.BlockSpec((1, HALF), lambda s: (0, 0)),
            pl.BlockSpec((D, D), lambda s: (0, 0)),
        ],
        out_specs=[pl.BlockSpec((SBLK, D), lambda s: (s, 0))] * 3,
        out_shape=[jax.ShapeDtypeStruct((S, D), bf16)] * 3,
        compiler_params=_cparams(("arbitrary",)),
        interpret=_INTERPRET,
    )(x2, inwT, inb, g1, b1, inv, pswap)

    ctx = pl.pallas_call(
        _attn_kernel,
        grid=(H // 2, S // QBLK),
        in_specs=[
            pl.BlockSpec((QBLK, 2 * HD), lambda hp, sq: (sq, hp)),
            pl.BlockSpec((S, 2 * HD), lambda hp, sq: (0, hp)),
            pl.BlockSpec((S, 2 * HD), lambda hp, sq: (0, hp)),
        ],
        out_specs=pl.BlockSpec((QBLK, 2 * HD), lambda hp, sq: (sq, hp)),
        out_shape=jax.ShapeDtypeStruct((S, D), bf16),
        compiler_params=_cparams(("arbitrary", "arbitrary")),
        interpret=_INTERPRET,
    )(q2, k2, v2)

    outwT = out_w.T.astype(bf16)
    outb = out_b.reshape(1, D)
    g2 = ln2_g.reshape(1, D)
    b2 = ln2_b.reshape(1, D)
    rwT = jnp.zeros((D, EPAD), f32).at[:, :E].set(router_w.T)

    x1, h2b, w = pl.pallas_call(
        _post_kernel,
        grid=(NSB,),
        in_specs=[
            pl.BlockSpec((SBLK, D), lambda s: (s, 0)),
            pl.BlockSpec((SBLK, D), lambda s: (s, 0)),
            pl.BlockSpec((D, D), lambda s: (0, 0)),
            pl.BlockSpec((1, D), lambda s: (0, 0)),
            pl.BlockSpec((1, D), lambda s: (0, 0)),
            pl.BlockSpec((1, D), lambda s: (0, 0)),
            pl.BlockSpec((D, EPAD), lambda s: (0, 0)),
        ],
        out_specs=[
            pl.BlockSpec((SBLK, D), lambda s: (s, 0)),
            pl.BlockSpec((SBLK, D), lambda s: (s, 0)),
            pl.BlockSpec((SBLK, EPAD), lambda s: (s, 0)),
        ],
        out_shape=[
            jax.ShapeDtypeStruct((S, D), f32),
            jax.ShapeDtypeStruct((S, D), bf16),
            jax.ShapeDtypeStruct((S, EPAD), f32),
        ],
        compiler_params=_cparams(("arbitrary",)),
        interpret=_INTERPRET,
    )(ctx, x2, outwT, outb, g2, b2, rwT)

    pk = pl.pallas_call(
        _route_kernel,
        in_specs=[pl.BlockSpec((S, EPAD), lambda: (0, 0))],
        out_specs=pl.BlockSpec((S, EPAD), lambda: (0, 0)),
        out_shape=jax.ShapeDtypeStruct((S, EPAD), f32),
        interpret=_INTERPRET,
    )(w)

    dl = pk[:, 0:1]
    dh = pk[:, 1:2]
    wlo = pk[:, 2:3]
    whi = pk[:, 3:4]
    be = pk[:NB, 4].astype(jnp.int32)
    nact = pk[0:1, 5].astype(jnp.int32)

    xdisp = pl.pallas_call(
        _dispatch_kernel,
        grid=(NB,),
        in_specs=[
            pl.BlockSpec((S, 1), lambda rb: (0, 0)),
            pl.BlockSpec((S, 1), lambda rb: (0, 0)),
            pl.BlockSpec((S, D), lambda rb: (0, 0)),
        ],
        out_specs=pl.BlockSpec((M, D), lambda rb: (rb, 0)),
        out_shape=jax.ShapeDtypeStruct((RMAX, D), bf16),
        compiler_params=_cparams(("arbitrary",)),
        interpret=_INTERPRET,
    )(dl, dh, h2b)

    gwb = gate_w.astype(bf16)
    vwb = val_w.astype(bf16)
    owb = wo_w.astype(bf16)
    gb3 = gate_b.reshape(E * 2, 1, FBLK)
    vb3 = val_b.reshape(E * 2, 1, FBLK)
    ob3 = wo_b.reshape(E, 1, D)

    ydisp = pl.pallas_call(
        _gmm_kernel,
        grid_spec=pltpu.PrefetchScalarGridSpec(
            num_scalar_prefetch=2,
            grid=(2, NB),
            in_specs=[
                pl.BlockSpec((M, D), lambda f, b, be_r, na_r: (b, 0)),
                pl.BlockSpec((1, FBLK, D), lambda f, b, be_r, na_r: (be_r[b], f, 0)),
                pl.BlockSpec((1, FBLK, D), lambda f, b, be_r, na_r: (be_r[b], f, 0)),
                pl.BlockSpec((1, D, FBLK), lambda f, b, be_r, na_r: (be_r[b], 0, f)),
                pl.BlockSpec((1, 1, FBLK), lambda f, b, be_r, na_r: (be_r[b] * 2 + f, 0, 0)),
                pl.BlockSpec((1, 1, FBLK), lambda f, b, be_r, na_r: (be_r[b] * 2 + f, 0, 0)),
                pl.BlockSpec((1, 1, D), lambda f, b, be_r, na_r: (be_r[b], 0, 0)),
            ],
            out_specs=pl.BlockSpec((1, M, D), lambda f, b, be_r, na_r: (f, b, 0)),
        ),
        out_shape=jax.ShapeDtypeStruct((2, RMAX, D), bf16),
        compiler_params=_cparams(("arbitrary", "arbitrary")),
        interpret=_INTERPRET,
    )(be, nact, xdisp, gwb, vwb, owb, gb3, vb3, ob3)

    out = pl.pallas_call(
        _comb_kernel,
        grid_spec=pltpu.PrefetchScalarGridSpec(
            num_scalar_prefetch=1,
            grid=(NB,),
            in_specs=[
                pl.BlockSpec((S, 1), lambda rb, na_r: (0, 0)),
                pl.BlockSpec((S, 1), lambda rb, na_r: (0, 0)),
                pl.BlockSpec((S, 1), lambda rb, na_r: (0, 0)),
                pl.BlockSpec((S, 1), lambda rb, na_r: (0, 0)),
                pl.BlockSpec((S, D), lambda rb, na_r: (0, 0)),
                pl.BlockSpec((1, M, D), lambda rb, na_r: (0, rb, 0)),
                pl.BlockSpec((1, M, D), lambda rb, na_r: (1, rb, 0)),
            ],
            out_specs=pl.BlockSpec((S, D), lambda rb, na_r: (0, 0)),
        ),
        out_shape=jax.ShapeDtypeStruct((S, D), f32),
        compiler_params=_cparams(("arbitrary",)),
        interpret=_INTERPRET,
    )(nact, dl, dh, wlo, whi, x1, ydisp, ydisp)

    return out.reshape(B, S, D)
